# Initial kernel scaffold; baseline (speedup 1.0000x reference)
import jax
import jax.numpy as jnp
from jax.experimental import pallas as pl


def kernel(user_emb, item_emb, edge_index, edge_vals):
    raise NotImplementedError("write your pallas kernel here")


# speedup vs baseline: 3.9063x; 3.9063x over previous
import functools

import jax
import jax.numpy as jnp
from jax import lax
from jax.experimental import pallas as pl
from jax.experimental.pallas import tpu as pltpu
from jax.experimental.pallas import tpu_sc as plsc

USER_NUM = 5000
ITEM_NUM = 5000
N_NODES = USER_NUM + ITEM_NUM
N_EDGES = 320000
EMB = 128
N_LAYERS = 3

NC = 2
NS = 16
NW = NC * NS
LANES = 16

CHUNK = 128
N_CHUNKS = N_EDGES // CHUNK
ZCH = 80
NZ = N_NODES // ZCH

RB = 40
NB_CHUNKS = N_NODES // RB

_mesh = plsc.VectorSubcoreMesh(core_axis_name="c", subcore_axis_name="s")


def _scatter_body(ego, row, col, vals, partials,
                  acc, colbuf, rowbuf, valbuf, rows, zbuf, gsem):
    c = lax.axis_index("c")
    s = lax.axis_index("s")
    w = s * NC + c

    def zero_body(r, carry):
        for k in range(EMB // LANES):
            zbuf[r, pl.ds(k * LANES, LANES)] = jnp.zeros((LANES,), jnp.float32)
        return carry
    lax.fori_loop(0, ZCH, zero_body, 0)
    nz = jnp.where(s < NZ % NS, NZ // NS + 1, NZ // NS)

    def zcopy_body(i, carry):
        pltpu.sync_copy(zbuf, acc.at[pl.ds((s + i * NS) * ZCH, ZCH)])
        return carry
    lax.fori_loop(0, nz, zcopy_body, 0)
    plsc.subcore_barrier()

    n_chunks = jnp.where(w < N_CHUNKS % NW, N_CHUNKS // NW + 1, N_CHUNKS // NW)

    def chunk_body(i, carry):
        base = (w + i * NW) * CHUNK
        pltpu.sync_copy(col.at[pl.ds(base, CHUNK)], colbuf)
        pltpu.sync_copy(row.at[pl.ds(base, CHUNK)], rowbuf)
        pltpu.sync_copy(vals.at[pl.ds(base, CHUNK)], valbuf)
        pltpu.async_copy(ego.at[colbuf], rows, gsem).wait()

        def edge_body(e, cc):
            vv = plsc.load_gather(valbuf, [jnp.full((LANES,), 0, jnp.int32) + e])
            for k in range(EMB // LANES):
                sl = pl.ds(k * LANES, LANES)
                rows[e, sl] = rows[e, sl] * vv
            return cc
        lax.fori_loop(0, CHUNK, edge_body, 0)

        pltpu.sync_copy(rows, acc.at[rowbuf], add=True)
        return carry
    lax.fori_loop(0, n_chunks, chunk_body, 0)

    plsc.subcore_barrier()

    def out_body(i, carry):
        sl = pl.ds((s + i * NS) * ZCH, ZCH)
        pltpu.sync_copy(acc.at[sl], partials.at[c, sl])
        return carry
    lax.fori_loop(0, nz, out_body, 0)


_scatter_layer = functools.partial(
    pl.kernel,
    mesh=_mesh,
    out_type=jax.ShapeDtypeStruct((NC, N_NODES, EMB), jnp.float32),
    scratch_types=[
        pltpu.VMEM_SHARED((N_NODES, EMB), jnp.float32),
        pltpu.VMEM((CHUNK,), jnp.int32),
        pltpu.VMEM((CHUNK,), jnp.int32),
        pltpu.VMEM((CHUNK,), jnp.float32),
        pltpu.VMEM((CHUNK, EMB), jnp.float32),
        pltpu.VMEM((ZCH, EMB), jnp.float32),
        pltpu.SemaphoreType.DMA,
    ],
    compiler_params=pltpu.CompilerParams(needs_layout_passes=False),
)(_scatter_body)


def _combine_body(partials, sum_in, ego_out, sum_out, p0, p1, sb):
    c = lax.axis_index("c")
    s = lax.axis_index("s")
    w = s * NC + c
    n = jnp.where(w < NB_CHUNKS % NW, NB_CHUNKS // NW + 1, NB_CHUNKS // NW)

    def body(i, carry):
        base = (w + i * NW) * RB
        pltpu.sync_copy(partials.at[0, pl.ds(base, RB)], p0)
        pltpu.sync_copy(partials.at[1, pl.ds(base, RB)], p1)
        pltpu.sync_copy(sum_in.at[pl.ds(base, RB)], sb)

        def rbody(r, cc):
            for k in range(EMB // LANES):
                sl = pl.ds(k * LANES, LANES)
                e = p0[r, sl] + p1[r, sl]
                p0[r, sl] = e
                sb[r, sl] = sb[r, sl] + e * (1.0 / 3.0)
            return cc
        lax.fori_loop(0, RB, rbody, 0)

        pltpu.sync_copy(p0, ego_out.at[pl.ds(base, RB)])
        pltpu.sync_copy(sb, sum_out.at[pl.ds(base, RB)])
        return carry
    lax.fori_loop(0, n, body, 0)


_combine_layer = functools.partial(
    pl.kernel,
    mesh=_mesh,
    out_type=(
        jax.ShapeDtypeStruct((N_NODES, EMB), jnp.float32),
        jax.ShapeDtypeStruct((N_NODES, EMB), jnp.float32),
    ),
    scratch_types=[
        pltpu.VMEM((RB, EMB), jnp.float32),
        pltpu.VMEM((RB, EMB), jnp.float32),
        pltpu.VMEM((RB, EMB), jnp.float32),
    ],
)(_combine_body)


def kernel(user_emb, item_emb, edge_index, edge_vals):
    ego = jnp.concatenate([user_emb, item_emb], axis=0)
    row = edge_index[0]
    col = edge_index[1]
    total = jnp.zeros((N_NODES, EMB), jnp.float32)
    for _ in range(N_LAYERS):
        partials = _scatter_layer(ego, row, col, edge_vals)
        ego, total = _combine_layer(partials, total)
    return (total[:USER_NUM], total[USER_NUM:])
